# baseline (speedup 1.0000x reference)
import functools

import jax
import jax.numpy as jnp
from jax import lax
from jax.experimental import pallas as pl
from jax.experimental.pallas import tpu as pltpu
from jax.experimental.pallas import tpu_sc as plsc

N_NODES = 10000
D = 128

NP = 10240
ROWS_PER_TILE = NP // 16
CHUNK = 128
CH_A = 92
CH_B = 66
E_PAD = 16 * (CH_A + CH_B) * CHUNK


def _sc_scatter(nfeat, src, dst):
    mesh = plsc.VectorSubcoreMesh(core_axis_name="c", subcore_axis_name="s")

    @functools.partial(
        pl.kernel,
        mesh=mesh,
        out_type=[
            jax.ShapeDtypeStruct((2, NP, D), jnp.float32),
            jax.ShapeDtypeStruct((2, NP), jnp.float32),
        ],
        scratch_types=[
            pltpu.VMEM((CHUNK,), jnp.int32),
            pltpu.VMEM((CHUNK,), jnp.int32),
            pltpu.VMEM((CHUNK, D), jnp.float32),
            pltpu.VMEM((CHUNK,), jnp.float32),
            pltpu.VMEM((16, D), jnp.float32),
            pltpu.VMEM((ROWS_PER_TILE,), jnp.float32),
            pltpu.VMEM_SHARED((NP, D), jnp.float32),
            pltpu.VMEM_SHARED((NP,), jnp.float32),
            pltpu.SemaphoreType.DMA,
        ],
    )
    def k(nfeat_hbm, src_hbm, dst_hbm, nsum_hbm, deg_hbm,
          src_v, dst_v, rows_v, ones_v, zrow_v, zdeg_v, acc_sh, deg_sh, sem):
        c = lax.axis_index("c")
        s = lax.axis_index("s")
        w = c * 16 + s

        zeros16 = jnp.zeros((16,), jnp.float32)
        for i in range(16):
            for j in range(D // 16):
                zrow_v[i, pl.ds(j * 16, 16)] = zeros16
        for j in range(D // 16):
            ones_v[pl.ds(j * 16, 16)] = jnp.ones((16,), jnp.float32)

        def zdeg_body(i, _):
            zdeg_v[pl.ds(i * 16, 16)] = zeros16
        lax.fori_loop(0, ROWS_PER_TILE // 16, zdeg_body, None)

        row0 = s * ROWS_PER_TILE

        def zacc_body(i, _):
            pltpu.sync_copy(zrow_v, acc_sh.at[pl.ds(row0 + i * 16, 16), :])
        lax.fori_loop(0, ROWS_PER_TILE // 16, zacc_body, None)
        pltpu.sync_copy(zdeg_v, deg_sh.at[pl.ds(row0, ROWS_PER_TILE)])
        plsc.subcore_barrier()

        n_my = jnp.where(c == 0, CH_A, CH_B)
        crow0 = jnp.where(c == 0, s * CH_A, 16 * CH_A + s * CH_B)
        base = crow0 * CHUNK

        def body(j, _):
            off = base + j * CHUNK
            pltpu.sync_copy(src_hbm.at[pl.ds(off, CHUNK)], src_v)
            pltpu.sync_copy(dst_hbm.at[pl.ds(off, CHUNK)], dst_v)
            pltpu.async_copy(nfeat_hbm.at[src_v], rows_v, sem).wait()
            pltpu.sync_copy(rows_v, acc_sh.at[dst_v], add=True)
            pltpu.sync_copy(ones_v, deg_sh.at[dst_v], add=True)
        lax.fori_loop(0, n_my, body, None)

        plsc.subcore_barrier()
        pltpu.sync_copy(acc_sh.at[pl.ds(row0, ROWS_PER_TILE), :],
                        nsum_hbm.at[c, pl.ds(row0, ROWS_PER_TILE), :])
        pltpu.sync_copy(deg_sh.at[pl.ds(row0, ROWS_PER_TILE)],
                        deg_hbm.at[c, pl.ds(row0, ROWS_PER_TILE)])

    return k(nfeat, src, dst)


def _tc_body(p0_ref, p1_ref, nf_ref, d0_ref, d1_ref, w_ref, b_ref, o_ref):
    h = p0_ref[...] + p1_ref[...] + 2.0 * nf_ref[...]
    m = lax.dot_general(h, w_ref[...], (((1,), (1,)), ((), ())),
                        preferred_element_type=jnp.float32)
    d = d0_ref[...] + d1_ref[...] + 1.0
    o_ref[...] = (m + b_ref[...]) / d + b_ref[...]


def _tc_finish(nsum, deg, nfeat, W, b):
    R = 1000
    grid = (N_NODES // R,)
    p0, p1 = nsum[0], nsum[1]
    d0 = deg[0].reshape(NP, 1)
    d1 = deg[1].reshape(NP, 1)
    b2 = b.reshape(1, D)
    return pl.pallas_call(
        _tc_body,
        grid=grid,
        in_specs=[
            pl.BlockSpec((R, D), lambda i: (i, 0)),
            pl.BlockSpec((R, D), lambda i: (i, 0)),
            pl.BlockSpec((R, D), lambda i: (i, 0)),
            pl.BlockSpec((R, 1), lambda i: (i, 0)),
            pl.BlockSpec((R, 1), lambda i: (i, 0)),
            pl.BlockSpec((D, D), lambda i: (0, 0)),
            pl.BlockSpec((1, D), lambda i: (0, 0)),
        ],
        out_specs=pl.BlockSpec((R, D), lambda i: (i, 0)),
        out_shape=jax.ShapeDtypeStruct((N_NODES, D), jnp.float32),
    )(p0, p1, nfeat, d0, d1, W, b2)


@jax.jit
def kernel(nfeat, edge_index, W_neigh, b_neigh):
    src = edge_index[0].astype(jnp.int32)
    dst = edge_index[1].astype(jnp.int32)
    n_edges = src.shape[0]
    pad = E_PAD - n_edges
    src = jnp.concatenate([src, jnp.zeros((pad,), jnp.int32)])
    dst = jnp.concatenate([dst, jnp.full((pad,), N_NODES, jnp.int32)])
    nsum, deg = _sc_scatter(nfeat, src, dst)
    return _tc_finish(nsum, deg, nfeat, W_neigh, b_neigh)


# speedup vs baseline: 1.0933x; 1.0933x over previous
import functools

import jax
import jax.numpy as jnp
from jax import lax
from jax.experimental import pallas as pl
from jax.experimental.pallas import tpu as pltpu
from jax.experimental.pallas import tpu_sc as plsc

N_NODES = 10000
D = 128

NP = 10240
ROWS_PER_TILE = NP // 16
CHUNK = 128
CH_A = 106
CH_B = 52
E_PAD = 16 * (CH_A + CH_B) * CHUNK


def _sc_scatter(nfeat, src, dst):
    mesh = plsc.VectorSubcoreMesh(core_axis_name="c", subcore_axis_name="s")

    @functools.partial(
        pl.kernel,
        mesh=mesh,
        out_type=[
            jax.ShapeDtypeStruct((2, NP, D), jnp.float32),
            jax.ShapeDtypeStruct((2, NP), jnp.float32),
        ],
        scratch_types=[
            pltpu.VMEM((CHUNK,), jnp.int32),
            pltpu.VMEM((CHUNK,), jnp.int32),
            pltpu.VMEM((CHUNK, D), jnp.float32),
            pltpu.VMEM((CHUNK,), jnp.float32),
            pltpu.VMEM((16, D), jnp.float32),
            pltpu.VMEM((ROWS_PER_TILE,), jnp.float32),
            pltpu.VMEM_SHARED((NP, D), jnp.float32),
            pltpu.VMEM_SHARED((NP,), jnp.float32),
            pltpu.SemaphoreType.DMA,
        ],
    )
    def k(nfeat_hbm, src_hbm, dst_hbm, nsum_hbm, deg_hbm,
          src_v, dst_v, rows_v, ones_v, zrow_v, zdeg_v, acc_sh, deg_sh, sem):
        c = lax.axis_index("c")
        s = lax.axis_index("s")
        w = c * 16 + s

        zeros16 = jnp.zeros((16,), jnp.float32)
        for i in range(16):
            for j in range(D // 16):
                zrow_v[i, pl.ds(j * 16, 16)] = zeros16
        for j in range(D // 16):
            ones_v[pl.ds(j * 16, 16)] = jnp.ones((16,), jnp.float32)

        def zdeg_body(i, _):
            zdeg_v[pl.ds(i * 16, 16)] = zeros16
        lax.fori_loop(0, ROWS_PER_TILE // 16, zdeg_body, None)

        row0 = s * ROWS_PER_TILE

        def zacc_body(i, _):
            pltpu.sync_copy(zrow_v, acc_sh.at[pl.ds(row0 + i * 16, 16), :])
        lax.fori_loop(0, ROWS_PER_TILE // 16, zacc_body, None)
        pltpu.sync_copy(zdeg_v, deg_sh.at[pl.ds(row0, ROWS_PER_TILE)])
        plsc.subcore_barrier()

        n_my = jnp.where(c == 0, CH_A, CH_B)
        crow0 = jnp.where(c == 0, s * CH_A, 16 * CH_A + s * CH_B)
        base = crow0 * CHUNK

        def body(j, _):
            off = base + j * CHUNK
            pltpu.sync_copy(src_hbm.at[pl.ds(off, CHUNK)], src_v)
            pltpu.sync_copy(dst_hbm.at[pl.ds(off, CHUNK)], dst_v)
            pltpu.async_copy(nfeat_hbm.at[src_v], rows_v, sem).wait()
            pltpu.sync_copy(rows_v, acc_sh.at[dst_v], add=True)
            pltpu.sync_copy(ones_v, deg_sh.at[dst_v], add=True)
        lax.fori_loop(0, n_my, body, None)

        plsc.subcore_barrier()
        pltpu.sync_copy(acc_sh.at[pl.ds(row0, ROWS_PER_TILE), :],
                        nsum_hbm.at[c, pl.ds(row0, ROWS_PER_TILE), :])
        pltpu.sync_copy(deg_sh.at[pl.ds(row0, ROWS_PER_TILE)],
                        deg_hbm.at[c, pl.ds(row0, ROWS_PER_TILE)])

    return k(nfeat, src, dst)


def _tc_body(p0_ref, p1_ref, nf_ref, d0_ref, d1_ref, w_ref, b_ref, o_ref):
    h = p0_ref[...] + p1_ref[...] + 2.0 * nf_ref[...]
    m = lax.dot_general(h, w_ref[...], (((1,), (1,)), ((), ())),
                        preferred_element_type=jnp.float32)
    d = d0_ref[...] + d1_ref[...] + 1.0
    o_ref[...] = (m + b_ref[...]) / d + b_ref[...]


def _tc_finish(nsum, deg, nfeat, W, b):
    R = 1000
    grid = (N_NODES // R,)
    p0, p1 = nsum[0], nsum[1]
    d0 = deg[0].reshape(NP, 1)
    d1 = deg[1].reshape(NP, 1)
    b2 = b.reshape(1, D)
    return pl.pallas_call(
        _tc_body,
        grid=grid,
        in_specs=[
            pl.BlockSpec((R, D), lambda i: (i, 0)),
            pl.BlockSpec((R, D), lambda i: (i, 0)),
            pl.BlockSpec((R, D), lambda i: (i, 0)),
            pl.BlockSpec((R, 1), lambda i: (i, 0)),
            pl.BlockSpec((R, 1), lambda i: (i, 0)),
            pl.BlockSpec((D, D), lambda i: (0, 0)),
            pl.BlockSpec((1, D), lambda i: (0, 0)),
        ],
        out_specs=pl.BlockSpec((R, D), lambda i: (i, 0)),
        out_shape=jax.ShapeDtypeStruct((N_NODES, D), jnp.float32),
    )(p0, p1, nfeat, d0, d1, W, b2)


@jax.jit
def kernel(nfeat, edge_index, W_neigh, b_neigh):
    src = edge_index[0].astype(jnp.int32)
    dst = edge_index[1].astype(jnp.int32)
    n_edges = src.shape[0]
    pad = E_PAD - n_edges
    src = jnp.concatenate([src, jnp.zeros((pad,), jnp.int32)])
    dst = jnp.concatenate([dst, jnp.full((pad,), N_NODES, jnp.int32)])
    nsum, deg = _sc_scatter(nfeat, src, dst)
    return _tc_finish(nsum, deg, nfeat, W_neigh, b_neigh)
